# baseline (speedup 1.0000x reference)
import jax
import jax.numpy as jnp
from jax import lax
from jax.experimental import pallas as pl
from jax.experimental.pallas import tpu as pltpu
from jax.experimental.pallas import tpu_sc as plsc

CAP = 100000
DIM = 64
BATCH = 16384
BATP = BATCH // 2
PDIM = 2 * DIM

NW = 32
SLAB = 3128
SLAB_LAST = CAP - (NW - 1) * SLAB
LANES = 16
POS_PAD = 3136
NG_SLAB = POS_PAD // LANES

BCH = 128
NBK = 23
TAIL_N = SLAB - 24 * BCH
TAIL_L = SLAB_LAST - 23 * BCH

GW = BCH + 8
LIST_SZ = 3264
NG_LIST = LIST_SZ // LANES

_INT_MIN = -2147483647 - 1


def _sc_body(mem_hbm, idx_hbm, val_hbm, out_hbm,
             idx_v, pos_v, row1_v, win1_v, wp1_v, vstage_v, cbuf_v,
             isem0, isem1, isem2, isem3, osem0, osem1, osem2, osem3,
             gsem0, gsem1):
    wid = lax.axis_index("s") * 2 + lax.axis_index("c")
    base = wid * SLAB
    is_last = wid == NW - 1
    slab_len = jnp.where(is_last, SLAB_LAST, SLAB)

    isems = (isem0, isem1, isem2, isem3)
    osems = (osem0, osem1, osem2, osem3)
    gsems = (gsem0, gsem1)

    def cin(c, n):
        b = c % 4
        return pltpu.make_async_copy(
            mem_hbm.at[pl.ds(base + c * BCH, n)],
            cbuf_v.at[b, pl.ds(0, n)], isems[b])

    def cout(c, n):
        b = c % 4
        return pltpu.make_async_copy(
            cbuf_v.at[b, pl.ds(0, n)],
            out_hbm.at[pl.ds(base + c * BCH, n)], osems[b])

    cin(0, BCH).start()
    cin(1, BCH).start()

    with jax.named_scope("stage_idx"):
        pltpu.sync_copy(idx_hbm, idx_v)

    iota = lax.iota(jnp.int32, LANES)
    neg1 = jnp.full((LANES,), -1, jnp.int32)

    with jax.named_scope("init_pos"):
        @pl.loop(0, POS_PAD, step=LANES)
        def _(off):
            pos_v[pl.ds(off, LANES)] = neg1

    with jax.named_scope("scan"):
        @pl.loop(0, BATCH, step=LANES)
        def _(off):
            v = idx_v[pl.ds(off, LANES)]
            loc = v - base
            m = (loc >= 0) & (loc < slab_len)
            loc = jnp.where(m, loc, 0)
            plsc.store_scatter(pos_v, [loc], iota + off, mask=m)

    def _extract(g, carry):
        cnt, blo, bhi = carry
        p = pos_v[pl.ds(g * LANES, LANES)]
        m = p >= 0
        rows = iota + g * LANES
        plsc.store_compressed(row1_v.at[pl.ds(cnt, LANES)], rows, mask=m)
        plsc.store_compressed(win1_v.at[pl.ds(cnt, LANES)], p, mask=m)
        npop = plsc.all_reduce_population_count(m)
        bk = g >> 3
        blo = blo + jnp.where(iota == bk, npop, 0)
        bhi = bhi + jnp.where(iota == bk - LANES, npop, 0)
        return cnt + jnp.max(npop), blo, bhi

    zeros = jnp.zeros((LANES,), jnp.int32)
    with jax.named_scope("extract"):
        cnt, blo, bhi = lax.fori_loop(
            0, NG_SLAB, _extract, (jnp.int32(0), zeros, zeros))

    ends_lo = plsc.cumsum(blo)
    tot_lo = jnp.max(jnp.where(iota == LANES - 1, ends_lo, 0))
    ends_hi = plsc.cumsum(bhi) + tot_lo
    starts_lo = ends_lo - blo
    starts_hi = ends_hi - bhi

    int_min = jnp.int32(_INT_MIN)

    def bucket_range(c):
        if c < LANES:
            s = jnp.max(jnp.where(iota == c, starts_lo, int_min))
            e = jnp.max(jnp.where(iota == c, ends_lo, int_min))
        else:
            s = jnp.max(jnp.where(iota == c - LANES, starts_hi, int_min))
            e = jnp.max(jnp.where(iota == c - LANES, ends_hi, int_min))
        return s, e

    with jax.named_scope("fill"):
        @pl.loop(0, NG_LIST)
        def _(g):
            lanepos = iota + g * LANES
            keep = lanepos < cnt
            w = win1_v[pl.ds(g * LANES, LANES)]
            wp1_v[pl.ds(g * LANES, LANES)] = jnp.where(
                keep, lax.shift_right_logical(w, 1), lanepos & (BATP - 1))

    def g_copy(c):
        b = c % 2
        s, _ = bucket_range(c)
        fl8 = pl.multiple_of(s & ~jnp.int32(7), 8)
        return pltpu.make_async_copy(
            val_hbm.at[wp1_v.at[pl.ds(fl8, GW)]], vstage_v.at[b], gsems[b])

    with jax.named_scope("gprime"):
        g_copy(0).start()
        g_copy(1).start()

    int_min_s = jnp.int32(_INT_MIN)

    def patch(c, n):
        b = c % 2
        bc = c % 4
        s, e = bucket_range(c)
        fl8 = s & ~jnp.int32(7)

        def body(ei, carry):
            al = ei & ~jnp.int32(15)
            m = iota == (ei - al)
            rowv = row1_v[pl.ds(al, LANES)]
            wv = win1_v[pl.ds(al, LANES)]
            loce = jnp.max(jnp.where(m, rowv, int_min_s)) - c * BCH
            pare = (jnp.max(jnp.where(m, wv, int_min_s)) & 1) * DIM
            vrowe = ei - fl8
            for k in range(DIM // LANES):
                cbuf_v[bc, loce, pl.ds(LANES * k, LANES)] = (
                    vstage_v[b, vrowe, pl.ds(pare + LANES * k, LANES)])
            return carry

        lax.fori_loop(s, e, body, jnp.int32(0))

    with jax.named_scope("drain"):
        for c in range(NBK):
            if c >= 2:
                cout(c - 2, BCH).wait()
            if c + 2 < NBK:
                cin(c + 2, BCH).start()
            elif c + 2 == NBK:
                @pl.when(is_last)
                def _():
                    cin(NBK, TAIL_L).start()

                @pl.when(jnp.logical_not(is_last))
                def _():
                    cin(NBK, BCH).start()
            else:
                @pl.when(jnp.logical_not(is_last))
                def _():
                    cin(NBK + 1, TAIL_N).start()
            cin(c, BCH).wait()
            g_copy(c).wait()
            patch(c, BCH)
            if c + 2 <= NBK:
                g_copy(c + 2).start()
            else:
                @pl.when(jnp.logical_not(is_last))
                def _():
                    g_copy(c + 2).start()
            cout(c, BCH).start()

        cout(NBK - 2, BCH).wait()
        cout(NBK - 1, BCH).wait()

        @pl.when(is_last)
        def _():
            cin(NBK, TAIL_L).wait()
            g_copy(NBK).wait()
            patch(NBK, TAIL_L)
            cout(NBK, TAIL_L).start()
            cout(NBK, TAIL_L).wait()

        @pl.when(jnp.logical_not(is_last))
        def _():
            cin(NBK, BCH).wait()
            g_copy(NBK).wait()
            patch(NBK, BCH)
            g_copy(NBK + 1).start()
            cout(NBK, BCH).start()
            cin(NBK + 1, TAIL_N).wait()
            g_copy(NBK + 1).wait()
            patch(NBK + 1, TAIL_N)
            cout(NBK, BCH).wait()
            cout(NBK + 1, TAIL_N).start()
            cout(NBK + 1, TAIL_N).wait()


@jax.jit
def _scatter_sc(mem, idx32, val2):
    mesh = plsc.VectorSubcoreMesh(core_axis_name="c", subcore_axis_name="s")
    kfn = pl.kernel(
        _sc_body,
        out_type=jax.ShapeDtypeStruct((CAP, DIM), mem.dtype),
        mesh=mesh,
        compiler_params=pltpu.CompilerParams(needs_layout_passes=False),
        scratch_types=[
            pltpu.VMEM((BATCH,), jnp.int32),
            pltpu.VMEM((POS_PAD,), jnp.int32),
            pltpu.VMEM((LIST_SZ,), jnp.int32),
            pltpu.VMEM((LIST_SZ,), jnp.int32),
            pltpu.VMEM((LIST_SZ,), jnp.int32),
            pltpu.VMEM((2, GW, PDIM), jnp.float32),
            pltpu.VMEM((4, BCH, DIM), jnp.float32),
            pltpu.SemaphoreType.DMA,
            pltpu.SemaphoreType.DMA,
            pltpu.SemaphoreType.DMA,
            pltpu.SemaphoreType.DMA,
            pltpu.SemaphoreType.DMA,
            pltpu.SemaphoreType.DMA,
            pltpu.SemaphoreType.DMA,
            pltpu.SemaphoreType.DMA,
            pltpu.SemaphoreType.DMA,
            pltpu.SemaphoreType.DMA,
        ],
    )
    return kfn(mem, idx32, val2)


def kernel(mem, idx, val):
    val2 = val.reshape(BATP, PDIM)
    return _scatter_sc(mem, idx.astype(jnp.int32), val2)


# speedup vs baseline: 1.0130x; 1.0130x over previous
import jax
import jax.numpy as jnp
from jax import lax
from jax.experimental import pallas as pl
from jax.experimental.pallas import tpu as pltpu
from jax.experimental.pallas import tpu_sc as plsc

CAP = 100000
DIM = 64
BATCH = 16384
BATP = BATCH // 2
PDIM = 2 * DIM

NW = 32
SLAB = 3128
SLAB_LAST = CAP - (NW - 1) * SLAB
LANES = 16
POS_PAD = 3136
NG_SLAB = POS_PAD // LANES

BCH = 128
NBK = 23
TAIL_N = SLAB - 24 * BCH
TAIL_L = SLAB_LAST - 23 * BCH

GW = BCH + 8
LIST_SZ = 3264
NG_LIST = LIST_SZ // LANES

_INT_MIN = -2147483647 - 1


def _sc_body(mem_hbm, idx_hbm, val_hbm, out_hbm,
             idx_v, pos_v, row1_v, win1_v, wp1_v, vstage_v, cbuf_v,
             isem0, isem1, isem2, isem3, osem0, osem1, osem2, osem3,
             gsem0, gsem1):
    wid = lax.axis_index("s") * 2 + lax.axis_index("c")
    base = wid * SLAB
    is_last = wid == NW - 1
    slab_len = jnp.where(is_last, SLAB_LAST, SLAB)

    isems = (isem0, isem1, isem2, isem3)
    osems = (osem0, osem1, osem2, osem3)
    gsems = (gsem0, gsem1)

    def cin(c, n):
        b = c % 4
        return pltpu.make_async_copy(
            mem_hbm.at[pl.ds(base + c * BCH, n)],
            cbuf_v.at[b, pl.ds(0, n)], isems[b])

    def cout(c, n):
        b = c % 4
        return pltpu.make_async_copy(
            cbuf_v.at[b, pl.ds(0, n)],
            out_hbm.at[pl.ds(base + c * BCH, n)], osems[b])

    cin(0, BCH).start()
    cin(1, BCH).start()

    with jax.named_scope("stage_idx"):
        pltpu.sync_copy(idx_hbm, idx_v)

    iota = lax.iota(jnp.int32, LANES)
    neg1 = jnp.full((LANES,), -1, jnp.int32)

    with jax.named_scope("init_pos"):
        @pl.loop(0, POS_PAD, step=LANES)
        def _(off):
            pos_v[pl.ds(off, LANES)] = neg1

    with jax.named_scope("scan"):
        @pl.loop(0, BATCH, step=LANES)
        def _(off):
            v = idx_v[pl.ds(off, LANES)]
            loc = v - base
            m = (loc >= 0) & (loc < slab_len)
            loc = jnp.where(m, loc, 0)
            plsc.store_scatter(pos_v, [loc], iota + off, mask=m)

    def _extract(g, carry):
        cnt, blo, bhi = carry
        p = pos_v[pl.ds(g * LANES, LANES)]
        m = p >= 0
        rows = iota + g * LANES
        plsc.store_compressed(row1_v.at[pl.ds(cnt, LANES)], rows, mask=m)
        plsc.store_compressed(win1_v.at[pl.ds(cnt, LANES)], p, mask=m)
        npop = plsc.all_reduce_population_count(m)
        bk = g >> 3
        blo = blo + jnp.where(iota == bk, npop, 0)
        bhi = bhi + jnp.where(iota == bk - LANES, npop, 0)
        return cnt + jnp.max(npop), blo, bhi

    zeros = jnp.zeros((LANES,), jnp.int32)
    with jax.named_scope("extract"):
        cnt, blo, bhi = lax.fori_loop(
            0, NG_SLAB, _extract, (jnp.int32(0), zeros, zeros))

    ends_lo = plsc.cumsum(blo)
    tot_lo = jnp.max(jnp.where(iota == LANES - 1, ends_lo, 0))
    ends_hi = plsc.cumsum(bhi) + tot_lo
    starts_lo = ends_lo - blo
    starts_hi = ends_hi - bhi

    int_min = jnp.int32(_INT_MIN)

    def bucket_range(c):
        if c < LANES:
            s = jnp.max(jnp.where(iota == c, starts_lo, int_min))
            e = jnp.max(jnp.where(iota == c, ends_lo, int_min))
        else:
            s = jnp.max(jnp.where(iota == c - LANES, starts_hi, int_min))
            e = jnp.max(jnp.where(iota == c - LANES, ends_hi, int_min))
        return s, e

    with jax.named_scope("fill"):
        @pl.loop(0, NG_LIST)
        def _(g):
            lanepos = iota + g * LANES
            keep = lanepos < cnt
            w = win1_v[pl.ds(g * LANES, LANES)]
            wp1_v[pl.ds(g * LANES, LANES)] = jnp.where(
                keep, lax.shift_right_logical(w, 1), lanepos & (BATP - 1))

    def g_copy(c):
        b = c % 2
        s, _ = bucket_range(c)
        fl8 = pl.multiple_of(s & ~jnp.int32(7), 8)
        return pltpu.make_async_copy(
            val_hbm.at[wp1_v.at[pl.ds(fl8, GW)]], vstage_v.at[b], gsems[b])

    with jax.named_scope("gprime"):
        g_copy(0).start()
        g_copy(1).start()

    def patch(c, n):
        b = c % 2
        bc = c % 4
        s, e = bucket_range(c)
        fl8 = s & ~jnp.int32(7)
        ng = (e - s + LANES - 1) >> 4

        @pl.loop(0, ng)
        def _(t):
            off = s + t * LANES
            lanepos = off + iota
            lmask = lanepos < e
            rowv = row1_v[pl.ds(off, LANES)]
            w = win1_v[pl.ds(off, LANES)]
            parbase = (w & 1) * DIM
            loc = jnp.where(lmask, rowv - c * BCH, 0)
            vrow = jnp.where(lmask, lanepos - fl8, 0)

            @plsc.parallel_loop(0, DIM, step=1)
            def _(col):
                x = plsc.load_gather(vstage_v.at[b], [vrow, parbase + col])
                plsc.store_scatter(cbuf_v.at[bc], [loc, zeros + col], x,
                                   mask=lmask)

    with jax.named_scope("drain"):
        for c in range(NBK):
            if c >= 2:
                cout(c - 2, BCH).wait()
            if c + 2 < NBK:
                cin(c + 2, BCH).start()
            elif c + 2 == NBK:
                @pl.when(is_last)
                def _():
                    cin(NBK, TAIL_L).start()

                @pl.when(jnp.logical_not(is_last))
                def _():
                    cin(NBK, BCH).start()
            else:
                @pl.when(jnp.logical_not(is_last))
                def _():
                    cin(NBK + 1, TAIL_N).start()
            cin(c, BCH).wait()
            g_copy(c).wait()
            patch(c, BCH)
            if c + 2 <= NBK:
                g_copy(c + 2).start()
            else:
                @pl.when(jnp.logical_not(is_last))
                def _():
                    g_copy(c + 2).start()
            cout(c, BCH).start()

        cout(NBK - 2, BCH).wait()
        cout(NBK - 1, BCH).wait()

        @pl.when(is_last)
        def _():
            cin(NBK, TAIL_L).wait()
            g_copy(NBK).wait()
            patch(NBK, TAIL_L)
            cout(NBK, TAIL_L).start()
            cout(NBK, TAIL_L).wait()

        @pl.when(jnp.logical_not(is_last))
        def _():
            cin(NBK, BCH).wait()
            g_copy(NBK).wait()
            patch(NBK, BCH)
            g_copy(NBK + 1).start()
            cout(NBK, BCH).start()
            cin(NBK + 1, TAIL_N).wait()
            g_copy(NBK + 1).wait()
            patch(NBK + 1, TAIL_N)
            cout(NBK, BCH).wait()
            cout(NBK + 1, TAIL_N).start()
            cout(NBK + 1, TAIL_N).wait()


@jax.jit
def _scatter_sc(mem, idx32, val2):
    mesh = plsc.VectorSubcoreMesh(core_axis_name="c", subcore_axis_name="s")
    kfn = pl.kernel(
        _sc_body,
        out_type=jax.ShapeDtypeStruct((CAP, DIM), mem.dtype),
        mesh=mesh,
        compiler_params=pltpu.CompilerParams(needs_layout_passes=False),
        scratch_types=[
            pltpu.VMEM((BATCH,), jnp.int32),
            pltpu.VMEM((POS_PAD,), jnp.int32),
            pltpu.VMEM((LIST_SZ,), jnp.int32),
            pltpu.VMEM((LIST_SZ,), jnp.int32),
            pltpu.VMEM((LIST_SZ,), jnp.int32),
            pltpu.VMEM((2, GW, PDIM), jnp.float32),
            pltpu.VMEM((4, BCH, DIM), jnp.float32),
            pltpu.SemaphoreType.DMA,
            pltpu.SemaphoreType.DMA,
            pltpu.SemaphoreType.DMA,
            pltpu.SemaphoreType.DMA,
            pltpu.SemaphoreType.DMA,
            pltpu.SemaphoreType.DMA,
            pltpu.SemaphoreType.DMA,
            pltpu.SemaphoreType.DMA,
            pltpu.SemaphoreType.DMA,
            pltpu.SemaphoreType.DMA,
        ],
    )
    return kfn(mem, idx32, val2)


def kernel(mem, idx, val):
    val2 = val.reshape(BATP, PDIM)
    return _scatter_sc(mem, idx.astype(jnp.int32), val2)
